# baseline (speedup 1.0000x reference)
import functools

import jax
import jax.numpy as jnp
from jax import lax
from jax.experimental import pallas as pl
from jax.experimental.pallas import tpu as pltpu
from jax.experimental.pallas import tpu_sc as plsc

N = 50000
E = 800000
EP = 819200
IN_DIM = 36
HID = 64
HEADS = 4
HD = 16
B = 128
R = 2000
G = N // R
_IT = False

f32 = jnp.float32
i32 = jnp.int32


def _fullspec(shape):
    return pl.BlockSpec(shape, lambda *a: tuple(0 for _ in shape))


def _smemspec():
    return pl.BlockSpec((1, 1), lambda *a: (0, 0), memory_space=pltpu.SMEM)


def _rowspec(bshape):
    nd = len(bshape)
    if nd == 2:
        return pl.BlockSpec(bshape, lambda i: (i, 0))
    return pl.BlockSpec(bshape, lambda i: (0, i, 0))


def _tkA_body(x_ref, degp_ref, w_ref, dinv_ref, h2s_ref):
    deg = degp_ref[0, :, 0:1] + degp_ref[1, :, 0:1] + 1.0
    dinv = 1.0 / jnp.sqrt(jnp.clip(deg, 1.0, None))
    h2 = jnp.dot(x_ref[...], w_ref[...], preferred_element_type=f32) * dinv
    dinv_ref[...] = dinv
    h2s_ref[0] = h2[:, :32]
    h2s_ref[1] = h2[:, 32:]


def _tkA(x, degp, w0):
    return pl.pallas_call(
        _tkA_body,
        grid=(G,),
        in_specs=[_rowspec((R, IN_DIM)), _rowspec((2, R, 8)), _fullspec((IN_DIM, HID))],
        out_specs=[_rowspec((R, 1)), _rowspec((2, R, 32))],
        out_shape=[jax.ShapeDtypeStruct((N, 1), f32),
                   jax.ShapeDtypeStruct((2, N, 32), f32)],
        interpret=_IT,
    )(x, degp, w0)


def _tkB_body(seg_ref, h2s_ref, dinv_ref, b_ref, w_ref, as_ref, ad_ref,
              g_ref, hgs_ref, apad_ref, eas_ref):
    h2 = jnp.concatenate([h2s_ref[0], h2s_ref[1]], axis=1)
    seg = jnp.concatenate([seg_ref[0], seg_ref[1]], axis=1)
    g = jnp.maximum(dinv_ref[...] * (seg + h2) + b_ref[...], 0.0)
    hg = jnp.dot(g, w_ref[...], preferred_element_type=f32)
    ys = hg * as_ref[...]
    yd = hg * ad_ref[...]
    asrc = jnp.concatenate(
        [jnp.sum(ys[:, h * HD:(h + 1) * HD], axis=1, keepdims=True) for h in range(HEADS)], axis=1)
    adst = jnp.concatenate(
        [jnp.sum(yd[:, h * HD:(h + 1) * HD], axis=1, keepdims=True) for h in range(HEADS)], axis=1)
    al = asrc + adst
    eas_ref[...] = jnp.exp(jnp.maximum(al, 0.2 * al))
    g_ref[...] = g
    hgs_ref[0] = hg[:, :32]
    hgs_ref[1] = hg[:, 32:]
    apad_ref[...] = jnp.concatenate([asrc, adst], axis=1)


def _tkB(seg, h2s, dinv, gcn_b, gat_w, a_s, a_d):
    return pl.pallas_call(
        _tkB_body,
        grid=(G,),
        in_specs=[_rowspec((2, R, 32)), _rowspec((2, R, 32)),
                  _rowspec((R, 1)), _fullspec((1, HID)), _fullspec((HID, HID)),
                  _fullspec((1, HID)), _fullspec((1, HID))],
        out_specs=[_rowspec((R, HID)), _rowspec((2, R, 32)),
                   _rowspec((R, 8)), _rowspec((R, 4))],
        out_shape=[jax.ShapeDtypeStruct((N, HID), f32),
                   jax.ShapeDtypeStruct((2, N, 32), f32),
                   jax.ShapeDtypeStruct((N, 8), f32),
                   jax.ShapeDtypeStruct((N, 4), f32)],
        interpret=_IT,
    )(seg, h2s, dinv, gcn_b.reshape(1, HID), gat_w,
      a_s.reshape(1, HID), a_d.reshape(1, HID))


def _expand4(v4, rows):
    return jnp.concatenate(
        [jnp.broadcast_to(v4[:, h:h + 1], (rows, HD)) for h in range(HEADS)], axis=1)


def _gat_assemble(g_ref, hgs_ref, gseg_ref, dnp_ref, eas_ref, gb_ref, xin_ref):
    hg = jnp.concatenate([hgs_ref[0], hgs_ref[1]], axis=1)
    gseg = jnp.concatenate([gseg_ref[0], gseg_ref[1]], axis=1)
    den = dnp_ref[0][:, 0:4] + dnp_ref[1][:, 0:4] + eas_ref[...] + 1e-16
    num = gseg + hg * _expand4(eas_ref[...], R)
    h = g_ref[...] + num / _expand4(den, R) + gb_ref[...]
    if xin_ref is not None:
        h = h + xin_ref[...]
    return h


def _tkC01_body(g_ref, hgs_ref, gseg_ref, dnp_ref, eas_ref, gb_ref,
                xin_ref, wn_ref, dinv_ref, h_ref, h2s_ref):
    h = _gat_assemble(g_ref, hgs_ref, gseg_ref, dnp_ref, eas_ref, gb_ref, xin_ref)
    h2 = jnp.dot(h, wn_ref[...], preferred_element_type=f32) * dinv_ref[...]
    h_ref[...] = h
    h2s_ref[0] = h2[:, :32]
    h2s_ref[1] = h2[:, 32:]


def _tkC01(g, hgs, gseg, dnp, eas, gat_b, xin, wnext, dinv):
    has_xin = xin is not None
    ins = [g, hgs, gseg, dnp, eas, gat_b.reshape(1, HID)]
    specs = [_rowspec((R, HID)), _rowspec((2, R, 32)),
             _rowspec((2, R, 32)), _rowspec((2, R, 8)), _rowspec((R, 4)),
             _fullspec((1, HID))]
    if has_xin:
        ins.append(xin)
        specs.append(_rowspec((R, HID)))
    ins += [wnext, dinv]
    specs += [_fullspec((HID, HID)), _rowspec((R, 1))]

    def body(*refs):
        if has_xin:
            (g_r, hg_r, gs_r, dn_r, ea_r, gb_r, xin_r, wn_r, dv_r,
             h_r, h2_r) = refs
        else:
            (g_r, hg_r, gs_r, dn_r, ea_r, gb_r, wn_r, dv_r,
             h_r, h2_r) = refs
            xin_r = None
        _tkC01_body(g_r, hg_r, gs_r, dn_r, ea_r, gb_r, xin_r, wn_r, dv_r,
                    h_r, h2_r)

    return pl.pallas_call(
        body,
        grid=(G,),
        in_specs=specs,
        out_specs=[_rowspec((R, HID)), _rowspec((2, R, 32))],
        out_shape=[jax.ShapeDtypeStruct((N, HID), f32),
                   jax.ShapeDtypeStruct((2, N, 32), f32)],
        interpret=_IT,
    )(*ins)


def _tkC2_body(g_ref, hgs_ref, gseg_ref, dnp_ref, eas_ref, gb_ref,
               w1_ref, b1_ref, w2_ref, b2_ref,
               h_ref, s_ref, smax_ref, mx_ref):
    i = pl.program_id(0)
    h = _gat_assemble(g_ref, hgs_ref, gseg_ref, dnp_ref, eas_ref, gb_ref, None)
    t = jnp.tanh(jnp.dot(h, w1_ref[...], preferred_element_type=f32) + b1_ref[...])
    s = jnp.dot(t, w2_ref[...], preferred_element_type=f32) + b2_ref[...]
    h_ref[...] = h
    s_ref[...] = s

    @pl.when(i == 0)
    def _():
        mx_ref[0, 0] = -1e30
    mx_ref[0, 0] = jnp.maximum(mx_ref[0, 0], jnp.max(s))

    @pl.when(i == G - 1)
    def _():
        smax_ref[0, 0] = mx_ref[0, 0]


def _tkC2(g, hgs, gseg, dnp, eas, gat_b, att_w1, att_b1, att_w2, att_b2):
    return pl.pallas_call(
        _tkC2_body,
        grid=(G,),
        in_specs=[_rowspec((R, HID)), _rowspec((2, R, 32)),
                  _rowspec((2, R, 32)), _rowspec((2, R, 8)), _rowspec((R, 4)),
                  _fullspec((1, HID)),
                  _fullspec((HID, 32)), _fullspec((1, 32)),
                  _fullspec((32, 1)), _fullspec((1, 1))],
        out_specs=[_rowspec((R, HID)), _rowspec((R, 1)), _smemspec()],
        out_shape=[jax.ShapeDtypeStruct((N, HID), f32),
                   jax.ShapeDtypeStruct((N, 1), f32),
                   jax.ShapeDtypeStruct((1, 1), f32)],
        scratch_shapes=[pltpu.SMEM((1, 1), f32)],
        interpret=_IT,
    )(g, hgs, gseg, dnp, eas, gat_b.reshape(1, HID),
      att_w1, att_b1.reshape(1, 32), att_w2, att_b2.reshape(1, 1))


def _tkD_body(s_ref, smax_ref, e_ref, z_ref, acc_ref):
    i = pl.program_id(0)
    e = jnp.exp(s_ref[...] - smax_ref[0, 0])
    e_ref[...] = e

    @pl.when(i == 0)
    def _():
        acc_ref[0, 0] = 0.0
    acc_ref[0, 0] = acc_ref[0, 0] + jnp.sum(e)

    @pl.when(i == G - 1)
    def _():
        z_ref[0, 0] = acc_ref[0, 0]


def _tkD(s, smax):
    return pl.pallas_call(
        _tkD_body,
        grid=(G,),
        in_specs=[_rowspec((R, 1)), _smemspec()],
        out_specs=[_rowspec((R, 1)), _smemspec()],
        out_shape=[jax.ShapeDtypeStruct((N, 1), f32),
                   jax.ShapeDtypeStruct((1, 1), f32)],
        scratch_shapes=[pltpu.SMEM((1, 1), f32)],
        interpret=_IT,
    )(s, smax)


def _tkE_body(h_ref, s_ref, smax_ref, b_ref, addp_ref, wsum_ref, cnt_ref,
              gsum_ref, a_acc, w_acc, c_acc, g_acc):
    i = pl.program_id(0)

    @pl.when(i == 0)
    def _():
        a_acc[...] = jnp.zeros_like(a_acc)
        w_acc[...] = jnp.zeros_like(w_acc)
        c_acc[...] = jnp.zeros_like(c_acc)
        g_acc[...] = jnp.zeros_like(g_acc)

    mask = (b_ref[...] == lax.broadcasted_iota(i32, (1, B), 1)).astype(f32)
    h = h_ref[...]
    e = jnp.exp(s_ref[...] - smax_ref[0, 0])
    dn = (((0,), (0,)), ((), ()))
    a_acc[...] = a_acc[...] + lax.dot_general(mask, h, dn, preferred_element_type=f32)
    w_acc[...] = w_acc[...] + lax.dot_general(mask, e * h, dn, preferred_element_type=f32)
    c_acc[...] = c_acc[...] + jnp.sum(mask, axis=0).reshape(B, 1)
    g_acc[...] = g_acc[...] + lax.dot_general(mask, e, dn, preferred_element_type=f32)

    @pl.when(i == G - 1)
    def _():
        addp_ref[...] = a_acc[...]
        wsum_ref[...] = w_acc[...]
        cnt_ref[...] = c_acc[...]
        gsum_ref[...] = g_acc[...]


def _tkE(h, s, smax, batch2):
    return pl.pallas_call(
        _tkE_body,
        grid=(G,),
        in_specs=[_rowspec((R, HID)), _rowspec((R, 1)), _smemspec(),
                  _rowspec((R, 1))],
        out_specs=[_fullspec((B, HID)), _fullspec((B, HID)),
                   _fullspec((B, 1)), _fullspec((B, 1))],
        out_shape=[jax.ShapeDtypeStruct((B, HID), f32),
                   jax.ShapeDtypeStruct((B, HID), f32),
                   jax.ShapeDtypeStruct((B, 1), f32),
                   jax.ShapeDtypeStruct((B, 1), f32)],
        scratch_shapes=[pltpu.VMEM((B, HID), f32), pltpu.VMEM((B, HID), f32),
                        pltpu.VMEM((B, 1), f32), pltpu.VMEM((B, 1), f32)],
        interpret=_IT,
    )(h, s, smax, batch2)


def _tkF_body(addp_ref, wsum_ref, cnt_ref, gsum_ref, mxp_ref, out_ref):
    addp = addp_ref[...]
    meanp = addp / jnp.clip(cnt_ref[...], 1.0, None)
    z = jnp.sum(gsum_ref[...])
    ge = wsum_ref[...] / (gsum_ref[...] + 1e-8 * z)
    maxp = jnp.max(mxp_ref[...], axis=0)
    out_ref[...] = jnp.concatenate([ge, meanp, maxp, addp], axis=1)


def _tkF(addp, wsum, cnt, gsum, mxp):
    return pl.pallas_call(
        _tkF_body,
        in_specs=[_fullspec((B, HID)), _fullspec((B, HID)), _fullspec((B, 1)),
                  _fullspec((B, 1)),
                  pl.BlockSpec((32, B, HID), lambda: (0, 0, 0))],
        out_specs=_fullspec((B, 4 * HID)),
        out_shape=jax.ShapeDtypeStruct((B, 4 * HID), f32),
        interpret=_IT,
    )(addp, wsum, cnt, gsum, mxp)


RACC = 50176
TROW = RACC // 16
ZR = TROW // 4
OR0 = 3128
OR15 = N - 15 * OR0
CH = 128
EW = EP // 32
NCH_W = EW // CH
ES = EP // 16
NCH_S = ES // CH
PW = 1568
NPAD = 32 * PW
KF = 16
KF2 = 8
KW = 8


def _sc_mesh():
    return plsc.VectorSubcoreMesh(core_axis_name="c", subcore_axis_name="s",
                                  num_cores=2, num_subcores=16)


def _zero_acc(z_h, acc, s):
    for kk in range(4):
        pltpu.sync_copy(z_h, acc.at[pl.ds(s * TROW + kk * ZR, ZR)])


def _copy_out(acc, out_h, c, s):
    @pl.when(s < 15)
    def _():
        pltpu.sync_copy(acc.at[pl.ds(s * OR0, OR0)],
                        out_h.at[c, pl.ds(s * OR0, OR0)])

    @pl.when(s == 15)
    def _():
        pltpu.sync_copy(acc.at[pl.ds(15 * OR0, OR15)],
                        out_h.at[c, pl.ds(15 * OR0, OR15)])


def _sc_deg(dstp, ones8, z8):
    @functools.partial(
        pl.kernel,
        out_type=jax.ShapeDtypeStruct((2, N, 8), f32),
        mesh=_sc_mesh(),
        compiler_params=pltpu.CompilerParams(use_tc_tiling_on_sc=False, needs_layout_passes=False),
        scratch_types=[pltpu.VMEM_SHARED((RACC, 8), f32),
                       pltpu.VMEM((CH,), i32),
                       pltpu.VMEM((CH, 8), f32)],
    )
    def k(dst_h, ones_h, z_h, out_h, acc, idxd, onesv):
        c = lax.axis_index("c")
        s = lax.axis_index("s")
        _zero_acc(z_h, acc, s)
        pltpu.sync_copy(ones_h, onesv)
        plsc.subcore_barrier()
        w = c * 16 + s

        def step(t, carry):
            off = w * EW + t * CH
            pltpu.sync_copy(dst_h.at[pl.ds(off, CH)], idxd)
            pltpu.sync_copy(onesv, acc.at[idxd], add=True)
            return carry

        lax.fori_loop(0, NCH_W, step, 0)
        plsc.subcore_barrier()
        _copy_out(acc, out_h, c, s)

    return k(dstp, ones8, z8)


def _sc_gcnseg(h2s, srcp2, dstp2, z32):
    @functools.partial(
        pl.kernel,
        out_type=jax.ShapeDtypeStruct((2, N, 32), f32),
        mesh=_sc_mesh(),
        compiler_params=pltpu.CompilerParams(use_tc_tiling_on_sc=False, needs_layout_passes=False),
        scratch_types=[pltpu.VMEM_SHARED((RACC, 32), f32),
                       pltpu.VMEM((KF, CH), i32),
                       pltpu.VMEM((KF, CH), i32),
                       pltpu.VMEM((CH, 32), f32),
                       pltpu.VMEM((CH, 32), f32),
                       pltpu.VMEM((CH, 32), f32),
                       pltpu.SemaphoreType.DMA,
                       pltpu.SemaphoreType.DMA,
                       pltpu.SemaphoreType.DMA],
    )
    def k(tbl_h, src_h, dst_h, z_h, out_h, acc, idxs2, idxd2, rows0, rows1,
          rows2, sem0, sem1, sem2):
        c = lax.axis_index("c")
        s = lax.axis_index("s")
        _zero_acc(z_h, acc, s)
        plsc.subcore_barrier()
        rows = [rows0, rows1, rows2]
        sems = [sem0, sem1, sem2]

        def outer(j, carry):
            r0 = s * NCH_S // KF * KF + j * KF
            pltpu.sync_copy(src_h.at[pl.ds(r0, KF)], idxs2)
            pltpu.sync_copy(dst_h.at[pl.ds(r0, KF)], idxd2)
            cps = [None] * KF
            for p in range(2):
                cps[p] = pltpu.async_copy(
                    tbl_h.at[c].at[idxs2.at[p]], rows[p % 3], sems[p % 3])
            for kk in range(KF):
                if kk + 2 < KF:
                    cps[kk + 2] = pltpu.async_copy(
                        tbl_h.at[c].at[idxs2.at[kk + 2]],
                        rows[(kk + 2) % 3], sems[(kk + 2) % 3])
                cps[kk].wait()
                pltpu.sync_copy(rows[kk % 3], acc.at[idxd2.at[kk]], add=True)
            return carry

        lax.fori_loop(0, NCH_S // KF, outer, 0)
        plsc.subcore_barrier()
        _copy_out(acc, out_h, c, s)

    return k(h2s, srcp2, dstp2, z32)


def _sc_gat1(apad_p, srcp2, dstp2, z8, zc8):
    @functools.partial(
        pl.kernel,
        out_type=[jax.ShapeDtypeStruct((2, N, 8), f32),
                  jax.ShapeDtypeStruct((EP, 4), f32)],
        mesh=_sc_mesh(),
        compiler_params=pltpu.CompilerParams(use_tc_tiling_on_sc=False, needs_layout_passes=False),
        scratch_types=[pltpu.VMEM_SHARED((RACC, 8), f32),
                       pltpu.VMEM((KW, CH), i32),
                       pltpu.VMEM((KW, CH), i32),
                       pltpu.VMEM((CH, 8), f32),
                       pltpu.VMEM((CH, 8), f32),
                       pltpu.VMEM((CH, 8), f32),
                       pltpu.VMEM((CH, 8), f32),
                       pltpu.VMEM((CH, 4), f32),
                       pltpu.VMEM((CH, 8), f32),
                       pltpu.SemaphoreType.DMA,
                       pltpu.SemaphoreType.DMA],
    )
    def k(apad_h, src_h, dst_h, z_h, zc_h, dnp_h, eap_h, acc, idxs2, idxd2,
          ar0, dr0, ar1, dr1, eab, eab8, semA0, semA1):
        c = lax.axis_index("c")
        s = lax.axis_index("s")
        _zero_acc(z_h, acc, s)
        pltpu.sync_copy(zc_h, eab8)
        plsc.subcore_barrier()
        iot = lax.iota(i32, 16)
        rq = lax.shift_right_logical(iot, 2)
        ca = jnp.bitwise_and(iot, 3)
        cd = ca + 4
        w = c * 16 + s
        ars = [ar0, ar1]
        drs = [dr0, dr1]
        sems = [semA0, semA1]

        def gath(kk, i2):
            b = kk % 2
            cpa = pltpu.async_copy(apad_h.at[i2.at[kk]], ars[b], sems[b])
            cpd = pltpu.async_copy(apad_h.at[idxd2.at[kk]], drs[b], sems[b])
            return cpa, cpd

        def outer(j, carry):
            r0 = w * NCH_W // KW * KW + j * KW
            pltpu.sync_copy(src_h.at[pl.ds(r0, KW)], idxs2)
            pltpu.sync_copy(dst_h.at[pl.ds(r0, KW)], idxd2)
            cps = gath(0, idxs2)
            for kk in range(KW):
                b = kk % 2
                if kk + 1 < KW:
                    nxt = gath(kk + 1, idxs2)
                cps[0].wait()
                cps[1].wait()
                arows = ars[b]
                drows = drs[b]

                def inner(q, c2):
                    r = rq + q * 4
                    av = plsc.load_gather(arows, [r, ca])
                    dv = plsc.load_gather(drows, [r, cd])
                    al = av + dv
                    ea = jnp.exp(jnp.maximum(al, 0.2 * al))
                    plsc.store_scatter(eab, [r, ca], ea)
                    plsc.store_scatter(eab8, [r, ca], ea)
                    return c2

                lax.fori_loop(0, CH // 4, inner, 0)
                off = (r0 + kk) * CH
                pltpu.sync_copy(eab, eap_h.at[pl.ds(off, CH)])
                pltpu.sync_copy(eab8, acc.at[idxd2.at[kk]], add=True)
                if kk + 1 < KW:
                    cps = nxt
            return carry

        lax.fori_loop(0, NCH_W // KW, outer, 0)
        plsc.subcore_barrier()
        _copy_out(acc, dnp_h, c, s)

    return k(apad_p, srcp2, dstp2, z8, zc8)


def _sc_gat2(hgs, eap, srcp2, dstp2, z32):
    @functools.partial(
        pl.kernel,
        out_type=jax.ShapeDtypeStruct((2, N, 32), f32),
        mesh=_sc_mesh(),
        compiler_params=pltpu.CompilerParams(use_tc_tiling_on_sc=False, needs_layout_passes=False),
        scratch_types=[pltpu.VMEM_SHARED((RACC, 32), f32),
                       pltpu.VMEM((KF2, CH), i32),
                       pltpu.VMEM((KF2, CH), i32),
                       pltpu.VMEM((CH, 32), f32),
                       pltpu.VMEM((CH, 32), f32),
                       pltpu.VMEM((CH, 32), f32),
                       pltpu.VMEM((KF2 * CH, 4), f32),
                       pltpu.SemaphoreType.DMA,
                       pltpu.SemaphoreType.DMA,
                       pltpu.SemaphoreType.DMA],
    )
    def k(tbl_h, eap_h, src_h, dst_h, z_h, out_h, acc, idxs2, idxd2,
          rows0, rows1, rows2, eab, sem0, sem1, sem2):
        c = lax.axis_index("c")
        s = lax.axis_index("s")
        _zero_acc(z_h, acc, s)
        plsc.subcore_barrier()
        iot = lax.iota(i32, 16)
        col0 = iot
        col1 = iot + 16
        ec0 = jnp.full((16,), 2 * c, i32)
        ec1 = ec0 + 1
        rows = [rows0, rows1, rows2]
        sems = [sem0, sem1, sem2]

        def outer(j, carry):
            r0 = s * NCH_S // KF2 * KF2 + j * KF2
            pltpu.sync_copy(src_h.at[pl.ds(r0, KF2)], idxs2)
            pltpu.sync_copy(dst_h.at[pl.ds(r0, KF2)], idxd2)
            pltpu.sync_copy(eap_h.at[pl.ds(r0 * CH, KF2 * CH)], eab)
            cps = [None] * KF2
            for p in range(2):
                cps[p] = pltpu.async_copy(
                    tbl_h.at[c].at[idxs2.at[p]], rows[p % 3], sems[p % 3])
            for kk in range(KF2):
                if kk + 2 < KF2:
                    cps[kk + 2] = pltpu.async_copy(
                        tbl_h.at[c].at[idxs2.at[kk + 2]],
                        rows[(kk + 2) % 3], sems[(kk + 2) % 3])
                cps[kk].wait()
                rb = rows[kk % 3]

                def inner(jj, c2):
                    j2 = jj * 2
                    for u in range(2):
                        bj = jnp.full((16,), j2 + u, i32)
                        be = bj + kk * CH
                        w0 = plsc.load_gather(eab, [be, ec0])
                        w1 = plsc.load_gather(eab, [be, ec1])
                        v0 = plsc.load_gather(rb, [bj, col0])
                        v1 = plsc.load_gather(rb, [bj, col1])
                        plsc.store_scatter(rb, [bj, col0], v0 * w0)
                        plsc.store_scatter(rb, [bj, col1], v1 * w1)
                    return c2

                lax.fori_loop(0, CH // 2, inner, 0)
                pltpu.sync_copy(rb, acc.at[idxd2.at[kk]], add=True)
            return carry

        lax.fori_loop(0, NCH_S // KF2, outer, 0)
        plsc.subcore_barrier()
        _copy_out(acc, out_h, c, s)

    return k(hgs, eap, srcp2, dstp2, z32)


def _sc_maxp(h_pad, batch_pad, neg_h):
    @functools.partial(
        pl.kernel,
        out_type=jax.ShapeDtypeStruct((32, B, HID), f32),
        mesh=_sc_mesh(),
        compiler_params=pltpu.CompilerParams(use_tc_tiling_on_sc=False, needs_layout_passes=False),
        scratch_types=[pltpu.VMEM((PW, HID), f32),
                       pltpu.VMEM((PW,), i32),
                       pltpu.VMEM((B, HID), f32)],
    )
    def k(h_h, bat_h, neg_hbm, out_h, hcv, batv, acc):
        c = lax.axis_index("c")
        s = lax.axis_index("s")
        w = c * 16 + s
        base = w * PW
        pltpu.sync_copy(h_h.at[pl.ds(base, PW)], hcv)
        pltpu.sync_copy(bat_h.at[pl.ds(base, PW)], batv)
        pltpu.sync_copy(neg_hbm, acc)
        iot = lax.iota(i32, 16)
        cols = [iot + 16 * q for q in range(4)]

        def step(n, carry):
            bn = jnp.full((16,), n, i32)
            bv = plsc.load_gather(batv, [bn])
            for q in range(4):
                hv = plsc.load_gather(hcv, [bn, cols[q]])
                cur = plsc.load_gather(acc, [bv, cols[q]])
                plsc.store_scatter(acc, [bv, cols[q]], jnp.maximum(cur, hv))
            return carry

        lax.fori_loop(0, PW, step, 0)
        pltpu.sync_copy(acc, out_h.at[w])

    return k(h_pad, batch_pad, neg_h)


def kernel(x, edge_index, batch,
           gcn_W0, gcn_b0, gat_W0, gat_as0, gat_ad0, gat_b0,
           gcn_W1, gcn_b1, gat_W1, gat_as1, gat_ad1, gat_b1,
           gcn_W2, gcn_b2, gat_W2, gat_as2, gat_ad2, gat_b2,
           att_W1, att_b1, att_W2, att_b2):
    pad = jnp.arange(EP - E, dtype=i32)
    srcp = jnp.concatenate([edge_index[0], pad % N])
    dstp = jnp.concatenate([edge_index[1], N + (pad % 128)])
    srcp2 = srcp.reshape(EP // CH, CH)
    dstp2 = dstp.reshape(EP // CH, CH)
    z32 = jnp.zeros((ZR, 32), f32)
    z8 = jnp.zeros((ZR, 8), f32)
    zc8 = jnp.zeros((CH, 8), f32)
    ones8 = jnp.ones((CH, 8), f32)
    negB = jnp.full((B, HID), -1e30, f32)

    degp = _sc_deg(dstp, ones8, z8)
    dinv, h2s = _tkA(x, degp, gcn_W0)

    params = [(gcn_W0, gcn_b0, gat_W0, gat_as0, gat_ad0, gat_b0),
              (gcn_W1, gcn_b1, gat_W1, gat_as1, gat_ad1, gat_b1),
              (gcn_W2, gcn_b2, gat_W2, gat_as2, gat_ad2, gat_b2)]

    xin = None
    for l in range(3):
        _, gcn_b, gat_w, a_s, a_d, gat_b = params[l]
        seg = _sc_gcnseg(h2s, srcp2, dstp2, z32)
        g, hgs, apad, eas = _tkB(seg, h2s, dinv, gcn_b, gat_w, a_s, a_d)
        apad_p = jnp.concatenate([apad, jnp.zeros((176, 8), f32)])
        dnp, eap = _sc_gat1(apad_p, srcp2, dstp2, z8, zc8)
        gseg = _sc_gat2(hgs, eap, srcp2, dstp2, z32)
        if l == 0:
            xin, h2s = _tkC01(g, hgs, gseg, dnp, eas, gat_b,
                              None, params[1][0], dinv)
        elif l == 1:
            _, h2s = _tkC01(g, hgs, gseg, dnp, eas, gat_b,
                            xin, params[2][0], dinv)
        else:
            h, s, smax = _tkC2(g, hgs, gseg, dnp, eas, gat_b,
                               att_W1, att_b1, att_W2, att_b2)

    addp, wsum, cnt, gsum = _tkE(h, s, smax, batch.reshape(N, 1))
    h_pad = jnp.concatenate([h, jnp.full((NPAD - N, HID), -1e30, f32)])
    batch_pad = jnp.concatenate([batch, jnp.full((NPAD - N,), B - 1, i32)])
    mxp = _sc_maxp(h_pad, batch_pad, negB)
    return _tkF(addp, wsum, cnt, gsum, mxp)


# speedup vs baseline: 1.0432x; 1.0432x over previous
import functools

import jax
import jax.numpy as jnp
from jax import lax
from jax.experimental import pallas as pl
from jax.experimental.pallas import tpu as pltpu
from jax.experimental.pallas import tpu_sc as plsc

N = 50000
E = 800000
EP = 819200
IN_DIM = 36
HID = 64
HEADS = 4
HD = 16
B = 128
R = 2000
G = N // R
_IT = False

f32 = jnp.float32
i32 = jnp.int32


def _fullspec(shape):
    return pl.BlockSpec(shape, lambda *a: tuple(0 for _ in shape))


def _smemspec():
    return pl.BlockSpec((1, 1), lambda *a: (0, 0), memory_space=pltpu.SMEM)


def _rowspec(bshape):
    nd = len(bshape)
    if nd == 2:
        return pl.BlockSpec(bshape, lambda i: (i, 0))
    return pl.BlockSpec(bshape, lambda i: (0, i, 0))


def _tkA_body(x_ref, degp_ref, w_ref, dinv_ref, h2s_ref):
    deg = degp_ref[0, :, 0:1] + degp_ref[1, :, 0:1] + 1.0
    dinv = 1.0 / jnp.sqrt(jnp.clip(deg, 1.0, None))
    h2 = jnp.dot(x_ref[...], w_ref[...], preferred_element_type=f32) * dinv
    dinv_ref[...] = dinv
    h2s_ref[0] = h2[:, :32]
    h2s_ref[1] = h2[:, 32:]


def _tkA(x, degp, w0):
    return pl.pallas_call(
        _tkA_body,
        grid=(G,),
        in_specs=[_rowspec((R, IN_DIM)), _rowspec((2, R, 8)), _fullspec((IN_DIM, HID))],
        out_specs=[_rowspec((R, 1)), _rowspec((2, R, 32))],
        out_shape=[jax.ShapeDtypeStruct((N, 1), f32),
                   jax.ShapeDtypeStruct((2, N, 32), f32)],
        interpret=_IT,
    )(x, degp, w0)


def _tkB_body(seg_ref, h2s_ref, dinv_ref, b_ref, w_ref, as_ref, ad_ref,
              g_ref, hgs_ref, apad_ref, eas_ref):
    h2 = jnp.concatenate([h2s_ref[0], h2s_ref[1]], axis=1)
    seg = jnp.concatenate([seg_ref[0], seg_ref[1]], axis=1)
    g = jnp.maximum(dinv_ref[...] * (seg + h2) + b_ref[...], 0.0)
    hg = jnp.dot(g, w_ref[...], preferred_element_type=f32)
    ys = hg * as_ref[...]
    yd = hg * ad_ref[...]
    asrc = jnp.concatenate(
        [jnp.sum(ys[:, h * HD:(h + 1) * HD], axis=1, keepdims=True) for h in range(HEADS)], axis=1)
    adst = jnp.concatenate(
        [jnp.sum(yd[:, h * HD:(h + 1) * HD], axis=1, keepdims=True) for h in range(HEADS)], axis=1)
    al = asrc + adst
    eas_ref[...] = jnp.exp(jnp.maximum(al, 0.2 * al))
    g_ref[...] = g
    hgs_ref[0] = hg[:, :32]
    hgs_ref[1] = hg[:, 32:]
    apad_ref[...] = jnp.concatenate([asrc, adst], axis=1)


def _tkB(seg, h2s, dinv, gcn_b, gat_w, a_s, a_d):
    return pl.pallas_call(
        _tkB_body,
        grid=(G,),
        in_specs=[_rowspec((2, R, 32)), _rowspec((2, R, 32)),
                  _rowspec((R, 1)), _fullspec((1, HID)), _fullspec((HID, HID)),
                  _fullspec((1, HID)), _fullspec((1, HID))],
        out_specs=[_rowspec((R, HID)), _rowspec((2, R, 32)),
                   _rowspec((R, 8)), _rowspec((R, 4))],
        out_shape=[jax.ShapeDtypeStruct((N, HID), f32),
                   jax.ShapeDtypeStruct((2, N, 32), f32),
                   jax.ShapeDtypeStruct((N, 8), f32),
                   jax.ShapeDtypeStruct((N, 4), f32)],
        interpret=_IT,
    )(seg, h2s, dinv, gcn_b.reshape(1, HID), gat_w,
      a_s.reshape(1, HID), a_d.reshape(1, HID))


def _expand4(v4, rows):
    return jnp.concatenate(
        [jnp.broadcast_to(v4[:, h:h + 1], (rows, HD)) for h in range(HEADS)], axis=1)


def _gat_assemble(g_ref, hgs_ref, gseg_ref, dnp_ref, eas_ref, gb_ref, xin_ref):
    hg = jnp.concatenate([hgs_ref[0], hgs_ref[1]], axis=1)
    gseg = jnp.concatenate([gseg_ref[0], gseg_ref[1]], axis=1)
    den = dnp_ref[0][:, 0:4] + dnp_ref[1][:, 0:4] + eas_ref[...] + 1e-16
    num = gseg + hg * _expand4(eas_ref[...], R)
    h = g_ref[...] + num / _expand4(den, R) + gb_ref[...]
    if xin_ref is not None:
        h = h + xin_ref[...]
    return h


def _tkC01_body(g_ref, hgs_ref, gseg_ref, dnp_ref, eas_ref, gb_ref,
                xin_ref, wn_ref, dinv_ref, h_ref, h2s_ref):
    h = _gat_assemble(g_ref, hgs_ref, gseg_ref, dnp_ref, eas_ref, gb_ref, xin_ref)
    h2 = jnp.dot(h, wn_ref[...], preferred_element_type=f32) * dinv_ref[...]
    h_ref[...] = h
    h2s_ref[0] = h2[:, :32]
    h2s_ref[1] = h2[:, 32:]


def _tkC01(g, hgs, gseg, dnp, eas, gat_b, xin, wnext, dinv):
    has_xin = xin is not None
    ins = [g, hgs, gseg, dnp, eas, gat_b.reshape(1, HID)]
    specs = [_rowspec((R, HID)), _rowspec((2, R, 32)),
             _rowspec((2, R, 32)), _rowspec((2, R, 8)), _rowspec((R, 4)),
             _fullspec((1, HID))]
    if has_xin:
        ins.append(xin)
        specs.append(_rowspec((R, HID)))
    ins += [wnext, dinv]
    specs += [_fullspec((HID, HID)), _rowspec((R, 1))]

    def body(*refs):
        if has_xin:
            (g_r, hg_r, gs_r, dn_r, ea_r, gb_r, xin_r, wn_r, dv_r,
             h_r, h2_r) = refs
        else:
            (g_r, hg_r, gs_r, dn_r, ea_r, gb_r, wn_r, dv_r,
             h_r, h2_r) = refs
            xin_r = None
        _tkC01_body(g_r, hg_r, gs_r, dn_r, ea_r, gb_r, xin_r, wn_r, dv_r,
                    h_r, h2_r)

    return pl.pallas_call(
        body,
        grid=(G,),
        in_specs=specs,
        out_specs=[_rowspec((R, HID)), _rowspec((2, R, 32))],
        out_shape=[jax.ShapeDtypeStruct((N, HID), f32),
                   jax.ShapeDtypeStruct((2, N, 32), f32)],
        interpret=_IT,
    )(*ins)


def _tkC2_body(g_ref, hgs_ref, gseg_ref, dnp_ref, eas_ref, gb_ref,
               w1_ref, b1_ref, w2_ref, b2_ref,
               h_ref, s_ref, smax_ref, mx_ref):
    i = pl.program_id(0)
    h = _gat_assemble(g_ref, hgs_ref, gseg_ref, dnp_ref, eas_ref, gb_ref, None)
    t = jnp.tanh(jnp.dot(h, w1_ref[...], preferred_element_type=f32) + b1_ref[...])
    s = jnp.dot(t, w2_ref[...], preferred_element_type=f32) + b2_ref[...]
    h_ref[...] = h
    s_ref[...] = s

    @pl.when(i == 0)
    def _():
        mx_ref[0, 0] = -1e30
    mx_ref[0, 0] = jnp.maximum(mx_ref[0, 0], jnp.max(s))

    @pl.when(i == G - 1)
    def _():
        smax_ref[0, 0] = mx_ref[0, 0]


def _tkC2(g, hgs, gseg, dnp, eas, gat_b, att_w1, att_b1, att_w2, att_b2):
    return pl.pallas_call(
        _tkC2_body,
        grid=(G,),
        in_specs=[_rowspec((R, HID)), _rowspec((2, R, 32)),
                  _rowspec((2, R, 32)), _rowspec((2, R, 8)), _rowspec((R, 4)),
                  _fullspec((1, HID)),
                  _fullspec((HID, 32)), _fullspec((1, 32)),
                  _fullspec((32, 1)), _fullspec((1, 1))],
        out_specs=[_rowspec((R, HID)), _rowspec((R, 1)), _smemspec()],
        out_shape=[jax.ShapeDtypeStruct((N, HID), f32),
                   jax.ShapeDtypeStruct((N, 1), f32),
                   jax.ShapeDtypeStruct((1, 1), f32)],
        scratch_shapes=[pltpu.SMEM((1, 1), f32)],
        interpret=_IT,
    )(g, hgs, gseg, dnp, eas, gat_b.reshape(1, HID),
      att_w1, att_b1.reshape(1, 32), att_w2, att_b2.reshape(1, 1))


def _tkD_body(s_ref, smax_ref, e_ref, z_ref, acc_ref):
    i = pl.program_id(0)
    e = jnp.exp(s_ref[...] - smax_ref[0, 0])
    e_ref[...] = e

    @pl.when(i == 0)
    def _():
        acc_ref[0, 0] = 0.0
    acc_ref[0, 0] = acc_ref[0, 0] + jnp.sum(e)

    @pl.when(i == G - 1)
    def _():
        z_ref[0, 0] = acc_ref[0, 0]


def _tkD(s, smax):
    return pl.pallas_call(
        _tkD_body,
        grid=(G,),
        in_specs=[_rowspec((R, 1)), _smemspec()],
        out_specs=[_rowspec((R, 1)), _smemspec()],
        out_shape=[jax.ShapeDtypeStruct((N, 1), f32),
                   jax.ShapeDtypeStruct((1, 1), f32)],
        scratch_shapes=[pltpu.SMEM((1, 1), f32)],
        interpret=_IT,
    )(s, smax)


def _tkE_body(h_ref, s_ref, smax_ref, b_ref, addp_ref, wsum_ref, cnt_ref,
              gsum_ref, a_acc, w_acc, c_acc, g_acc):
    i = pl.program_id(0)

    @pl.when(i == 0)
    def _():
        a_acc[...] = jnp.zeros_like(a_acc)
        w_acc[...] = jnp.zeros_like(w_acc)
        c_acc[...] = jnp.zeros_like(c_acc)
        g_acc[...] = jnp.zeros_like(g_acc)

    mask = (b_ref[...] == lax.broadcasted_iota(i32, (1, B), 1)).astype(f32)
    h = h_ref[...]
    e = jnp.exp(s_ref[...] - smax_ref[0, 0])
    dn = (((0,), (0,)), ((), ()))
    a_acc[...] = a_acc[...] + lax.dot_general(mask, h, dn, preferred_element_type=f32)
    w_acc[...] = w_acc[...] + lax.dot_general(mask, e * h, dn, preferred_element_type=f32)
    c_acc[...] = c_acc[...] + jnp.sum(mask, axis=0).reshape(B, 1)
    g_acc[...] = g_acc[...] + lax.dot_general(mask, e, dn, preferred_element_type=f32)

    @pl.when(i == G - 1)
    def _():
        addp_ref[...] = a_acc[...]
        wsum_ref[...] = w_acc[...]
        cnt_ref[...] = c_acc[...]
        gsum_ref[...] = g_acc[...]


def _tkE(h, s, smax, batch2):
    return pl.pallas_call(
        _tkE_body,
        grid=(G,),
        in_specs=[_rowspec((R, HID)), _rowspec((R, 1)), _smemspec(),
                  _rowspec((R, 1))],
        out_specs=[_fullspec((B, HID)), _fullspec((B, HID)),
                   _fullspec((B, 1)), _fullspec((B, 1))],
        out_shape=[jax.ShapeDtypeStruct((B, HID), f32),
                   jax.ShapeDtypeStruct((B, HID), f32),
                   jax.ShapeDtypeStruct((B, 1), f32),
                   jax.ShapeDtypeStruct((B, 1), f32)],
        scratch_shapes=[pltpu.VMEM((B, HID), f32), pltpu.VMEM((B, HID), f32),
                        pltpu.VMEM((B, 1), f32), pltpu.VMEM((B, 1), f32)],
        interpret=_IT,
    )(h, s, smax, batch2)


def _tkF_body(addp_ref, wsum_ref, cnt_ref, gsum_ref, mxp_ref, out_ref):
    addp = addp_ref[...]
    meanp = addp / jnp.clip(cnt_ref[...], 1.0, None)
    z = jnp.sum(gsum_ref[...])
    ge = wsum_ref[...] / (gsum_ref[...] + 1e-8 * z)
    maxp = jnp.max(mxp_ref[...], axis=0)
    out_ref[...] = jnp.concatenate([ge, meanp, maxp, addp], axis=1)


def _tkF(addp, wsum, cnt, gsum, mxp):
    return pl.pallas_call(
        _tkF_body,
        in_specs=[_fullspec((B, HID)), _fullspec((B, HID)), _fullspec((B, 1)),
                  _fullspec((B, 1)),
                  pl.BlockSpec((32, B, HID), lambda: (0, 0, 0))],
        out_specs=_fullspec((B, 4 * HID)),
        out_shape=jax.ShapeDtypeStruct((B, 4 * HID), f32),
        interpret=_IT,
    )(addp, wsum, cnt, gsum, mxp)


RACC = 50176
TROW = RACC // 16
ZR = TROW // 4
OR0 = 3128
OR15 = N - 15 * OR0
CH = 128
EW = EP // 32
NCH_W = EW // CH
ES = EP // 16
NCH_S = ES // CH
PW = 1568
NPAD = 32 * PW
KF = 16
KF2 = 16
KW = 8


def _sc_mesh():
    return plsc.VectorSubcoreMesh(core_axis_name="c", subcore_axis_name="s",
                                  num_cores=2, num_subcores=16)


def _zero_acc(z_h, acc, s):
    for kk in range(4):
        pltpu.sync_copy(z_h, acc.at[pl.ds(s * TROW + kk * ZR, ZR)])


def _copy_out(acc, out_h, c, s):
    @pl.when(s < 15)
    def _():
        pltpu.sync_copy(acc.at[pl.ds(s * OR0, OR0)],
                        out_h.at[c, pl.ds(s * OR0, OR0)])

    @pl.when(s == 15)
    def _():
        pltpu.sync_copy(acc.at[pl.ds(15 * OR0, OR15)],
                        out_h.at[c, pl.ds(15 * OR0, OR15)])


def _sc_deg(dstp, ones8, z8):
    @functools.partial(
        pl.kernel,
        out_type=jax.ShapeDtypeStruct((2, N, 8), f32),
        mesh=_sc_mesh(),
        compiler_params=pltpu.CompilerParams(use_tc_tiling_on_sc=False, needs_layout_passes=False),
        scratch_types=[pltpu.VMEM_SHARED((RACC, 8), f32),
                       pltpu.VMEM((CH,), i32),
                       pltpu.VMEM((CH, 8), f32)],
    )
    def k(dst_h, ones_h, z_h, out_h, acc, idxd, onesv):
        c = lax.axis_index("c")
        s = lax.axis_index("s")
        _zero_acc(z_h, acc, s)
        pltpu.sync_copy(ones_h, onesv)
        plsc.subcore_barrier()
        w = c * 16 + s

        def step(t, carry):
            off = w * EW + t * CH
            pltpu.sync_copy(dst_h.at[pl.ds(off, CH)], idxd)
            pltpu.sync_copy(onesv, acc.at[idxd], add=True)
            return carry

        lax.fori_loop(0, NCH_W, step, 0)
        plsc.subcore_barrier()
        _copy_out(acc, out_h, c, s)

    return k(dstp, ones8, z8)


def _sc_gcnseg(h2s, srcp2, dstp2, z32):
    @functools.partial(
        pl.kernel,
        out_type=jax.ShapeDtypeStruct((2, N, 32), f32),
        mesh=_sc_mesh(),
        compiler_params=pltpu.CompilerParams(use_tc_tiling_on_sc=False, needs_layout_passes=False),
        scratch_types=[pltpu.VMEM_SHARED((RACC, 32), f32),
                       pltpu.VMEM((KF, CH), i32),
                       pltpu.VMEM((KF, CH), i32),
                       pltpu.VMEM((CH, 32), f32),
                       pltpu.VMEM((CH, 32), f32),
                       pltpu.VMEM((CH, 32), f32),
                       pltpu.SemaphoreType.DMA,
                       pltpu.SemaphoreType.DMA,
                       pltpu.SemaphoreType.DMA],
    )
    def k(tbl_h, src_h, dst_h, z_h, out_h, acc, idxs2, idxd2, rows0, rows1,
          rows2, sem0, sem1, sem2):
        c = lax.axis_index("c")
        s = lax.axis_index("s")
        _zero_acc(z_h, acc, s)
        plsc.subcore_barrier()
        rows = [rows0, rows1, rows2]
        sems = [sem0, sem1, sem2]

        def outer(j, carry):
            r0 = s * NCH_S // KF * KF + j * KF
            pltpu.sync_copy(src_h.at[pl.ds(r0, KF)], idxs2)
            pltpu.sync_copy(dst_h.at[pl.ds(r0, KF)], idxd2)
            cps = [None] * KF
            for p in range(2):
                cps[p] = pltpu.async_copy(
                    tbl_h.at[c].at[idxs2.at[p]], rows[p % 3], sems[p % 3])
            for kk in range(KF):
                if kk + 2 < KF:
                    cps[kk + 2] = pltpu.async_copy(
                        tbl_h.at[c].at[idxs2.at[kk + 2]],
                        rows[(kk + 2) % 3], sems[(kk + 2) % 3])
                cps[kk].wait()
                pltpu.sync_copy(rows[kk % 3], acc.at[idxd2.at[kk]], add=True)
            return carry

        lax.fori_loop(0, NCH_S // KF, outer, 0)
        plsc.subcore_barrier()
        _copy_out(acc, out_h, c, s)

    return k(h2s, srcp2, dstp2, z32)


def _sc_gat1(apad_p, srcp2, dstp2, z8, zc8):
    @functools.partial(
        pl.kernel,
        out_type=[jax.ShapeDtypeStruct((2, N, 8), f32),
                  jax.ShapeDtypeStruct((EP, 4), f32)],
        mesh=_sc_mesh(),
        compiler_params=pltpu.CompilerParams(use_tc_tiling_on_sc=False, needs_layout_passes=False),
        scratch_types=[pltpu.VMEM_SHARED((RACC, 8), f32),
                       pltpu.VMEM((KW, CH), i32),
                       pltpu.VMEM((KW, CH), i32),
                       pltpu.VMEM((CH, 8), f32),
                       pltpu.VMEM((CH, 8), f32),
                       pltpu.VMEM((CH, 8), f32),
                       pltpu.VMEM((CH, 8), f32),
                       pltpu.VMEM((CH, 4), f32),
                       pltpu.VMEM((CH, 8), f32),
                       pltpu.SemaphoreType.DMA,
                       pltpu.SemaphoreType.DMA],
    )
    def k(apad_h, src_h, dst_h, z_h, zc_h, dnp_h, eap_h, acc, idxs2, idxd2,
          ar0, dr0, ar1, dr1, eab, eab8, semA0, semA1):
        c = lax.axis_index("c")
        s = lax.axis_index("s")
        _zero_acc(z_h, acc, s)
        pltpu.sync_copy(zc_h, eab8)
        plsc.subcore_barrier()
        iot = lax.iota(i32, 16)
        rq = lax.shift_right_logical(iot, 2)
        ca = jnp.bitwise_and(iot, 3)
        cd = ca + 4
        w = c * 16 + s
        ars = [ar0, ar1]
        drs = [dr0, dr1]
        sems = [semA0, semA1]

        def gath(kk, i2):
            b = kk % 2
            cpa = pltpu.async_copy(apad_h.at[i2.at[kk]], ars[b], sems[b])
            cpd = pltpu.async_copy(apad_h.at[idxd2.at[kk]], drs[b], sems[b])
            return cpa, cpd

        def outer(j, carry):
            r0 = w * NCH_W // KW * KW + j * KW
            pltpu.sync_copy(src_h.at[pl.ds(r0, KW)], idxs2)
            pltpu.sync_copy(dst_h.at[pl.ds(r0, KW)], idxd2)
            cps = gath(0, idxs2)
            for kk in range(KW):
                b = kk % 2
                if kk + 1 < KW:
                    nxt = gath(kk + 1, idxs2)
                cps[0].wait()
                cps[1].wait()
                arows = ars[b]
                drows = drs[b]

                def inner(q, c2):
                    r = rq + q * 4
                    av = plsc.load_gather(arows, [r, ca])
                    dv = plsc.load_gather(drows, [r, cd])
                    al = av + dv
                    ea = jnp.exp(jnp.maximum(al, 0.2 * al))
                    plsc.store_scatter(eab, [r, ca], ea)
                    plsc.store_scatter(eab8, [r, ca], ea)
                    return c2

                lax.fori_loop(0, CH // 4, inner, 0)
                off = (r0 + kk) * CH
                pltpu.sync_copy(eab, eap_h.at[pl.ds(off, CH)])
                pltpu.sync_copy(eab8, acc.at[idxd2.at[kk]], add=True)
                if kk + 1 < KW:
                    cps = nxt
            return carry

        lax.fori_loop(0, NCH_W // KW, outer, 0)
        plsc.subcore_barrier()
        _copy_out(acc, dnp_h, c, s)

    return k(apad_p, srcp2, dstp2, z8, zc8)


def _sc_gat2(hgs, eap, srcp2, dstp2, z32):
    @functools.partial(
        pl.kernel,
        out_type=jax.ShapeDtypeStruct((2, N, 32), f32),
        mesh=_sc_mesh(),
        compiler_params=pltpu.CompilerParams(use_tc_tiling_on_sc=False, needs_layout_passes=False),
        scratch_types=[pltpu.VMEM_SHARED((RACC, 32), f32),
                       pltpu.VMEM((KF2, CH), i32),
                       pltpu.VMEM((KF2, CH), i32),
                       pltpu.VMEM((CH, 32), f32),
                       pltpu.VMEM((CH, 32), f32),
                       pltpu.VMEM((KF2 * CH, 4), f32),
                       pltpu.SemaphoreType.DMA,
                       pltpu.SemaphoreType.DMA],
    )
    def k(tbl_h, eap_h, src_h, dst_h, z_h, out_h, acc, idxs2, idxd2,
          rows0, rows1, eab, sem0, sem1):
        c = lax.axis_index("c")
        s = lax.axis_index("s")
        _zero_acc(z_h, acc, s)
        plsc.subcore_barrier()
        iot = lax.iota(i32, 16)
        col0 = iot
        col1 = iot + 16
        ec0 = jnp.full((16,), 2 * c, i32)
        ec1 = ec0 + 1
        rows = [rows0, rows1]
        sems = [sem0, sem1]

        def outer(j, carry):
            r0 = s * NCH_S // KF2 * KF2 + j * KF2
            pltpu.sync_copy(src_h.at[pl.ds(r0, KF2)], idxs2)
            pltpu.sync_copy(dst_h.at[pl.ds(r0, KF2)], idxd2)
            pltpu.sync_copy(eap_h.at[pl.ds(r0 * CH, KF2 * CH)], eab)
            cp = pltpu.async_copy(tbl_h.at[c].at[idxs2.at[0]], rows0, sem0)
            for kk in range(KF2):
                if kk + 1 < KF2:
                    nxt = pltpu.async_copy(
                        tbl_h.at[c].at[idxs2.at[kk + 1]],
                        rows[(kk + 1) % 2], sems[(kk + 1) % 2])
                cp.wait()
                rb = rows[kk % 2]

                def inner(jj, c2):
                    bj = jnp.full((16,), jj, i32)
                    be = bj + kk * CH
                    w0 = plsc.load_gather(eab, [be, ec0])
                    w1 = plsc.load_gather(eab, [be, ec1])
                    v0 = plsc.load_gather(rb, [bj, col0])
                    v1 = plsc.load_gather(rb, [bj, col1])
                    plsc.store_scatter(rb, [bj, col0], v0 * w0)
                    plsc.store_scatter(rb, [bj, col1], v1 * w1)
                    return c2

                lax.fori_loop(0, CH, inner, 0)
                pltpu.sync_copy(rb, acc.at[idxd2.at[kk]], add=True)
                if kk + 1 < KF2:
                    cp = nxt
            return carry

        lax.fori_loop(0, NCH_S // KF2, outer, 0)
        plsc.subcore_barrier()
        _copy_out(acc, out_h, c, s)

    return k(hgs, eap, srcp2, dstp2, z32)


def _sc_maxp(h_pad, batch_pad, neg_h):
    @functools.partial(
        pl.kernel,
        out_type=jax.ShapeDtypeStruct((32, B, HID), f32),
        mesh=_sc_mesh(),
        compiler_params=pltpu.CompilerParams(use_tc_tiling_on_sc=False, needs_layout_passes=False),
        scratch_types=[pltpu.VMEM((PW, HID), f32),
                       pltpu.VMEM((PW,), i32),
                       pltpu.VMEM((B, HID), f32)],
    )
    def k(h_h, bat_h, neg_hbm, out_h, hcv, batv, acc):
        c = lax.axis_index("c")
        s = lax.axis_index("s")
        w = c * 16 + s
        base = w * PW
        pltpu.sync_copy(h_h.at[pl.ds(base, PW)], hcv)
        pltpu.sync_copy(bat_h.at[pl.ds(base, PW)], batv)
        pltpu.sync_copy(neg_hbm, acc)
        iot = lax.iota(i32, 16)
        cols = [iot + 16 * q for q in range(4)]

        def step(n, carry):
            bn = jnp.full((16,), n, i32)
            bv = plsc.load_gather(batv, [bn])
            for q in range(4):
                hv = plsc.load_gather(hcv, [bn, cols[q]])
                cur = plsc.load_gather(acc, [bv, cols[q]])
                plsc.store_scatter(acc, [bv, cols[q]], jnp.maximum(cur, hv))
            return carry

        lax.fori_loop(0, PW, step, 0)
        pltpu.sync_copy(acc, out_h.at[w])

    return k(h_pad, batch_pad, neg_h)


def kernel(x, edge_index, batch,
           gcn_W0, gcn_b0, gat_W0, gat_as0, gat_ad0, gat_b0,
           gcn_W1, gcn_b1, gat_W1, gat_as1, gat_ad1, gat_b1,
           gcn_W2, gcn_b2, gat_W2, gat_as2, gat_ad2, gat_b2,
           att_W1, att_b1, att_W2, att_b2):
    pad = jnp.arange(EP - E, dtype=i32)
    srcp = jnp.concatenate([edge_index[0], pad % N])
    dstp = jnp.concatenate([edge_index[1], N + (pad % 128)])
    srcp2 = srcp.reshape(EP // CH, CH)
    dstp2 = dstp.reshape(EP // CH, CH)
    z32 = jnp.zeros((ZR, 32), f32)
    z8 = jnp.zeros((ZR, 8), f32)
    zc8 = jnp.zeros((CH, 8), f32)
    ones8 = jnp.ones((CH, 8), f32)
    negB = jnp.full((B, HID), -1e30, f32)

    degp = _sc_deg(dstp, ones8, z8)
    dinv, h2s = _tkA(x, degp, gcn_W0)

    params = [(gcn_W0, gcn_b0, gat_W0, gat_as0, gat_ad0, gat_b0),
              (gcn_W1, gcn_b1, gat_W1, gat_as1, gat_ad1, gat_b1),
              (gcn_W2, gcn_b2, gat_W2, gat_as2, gat_ad2, gat_b2)]

    xin = None
    for l in range(3):
        _, gcn_b, gat_w, a_s, a_d, gat_b = params[l]
        seg = _sc_gcnseg(h2s, srcp2, dstp2, z32)
        g, hgs, apad, eas = _tkB(seg, h2s, dinv, gcn_b, gat_w, a_s, a_d)
        apad_p = jnp.concatenate([apad, jnp.zeros((176, 8), f32)])
        dnp, eap = _sc_gat1(apad_p, srcp2, dstp2, z8, zc8)
        gseg = _sc_gat2(hgs, eap, srcp2, dstp2, z32)
        if l == 0:
            xin, h2s = _tkC01(g, hgs, gseg, dnp, eas, gat_b,
                              None, params[1][0], dinv)
        elif l == 1:
            _, h2s = _tkC01(g, hgs, gseg, dnp, eas, gat_b,
                            xin, params[2][0], dinv)
        else:
            h, s, smax = _tkC2(g, hgs, gseg, dnp, eas, gat_b,
                               att_W1, att_b1, att_W2, att_b2)

    addp, wsum, cnt, gsum = _tkE(h, s, smax, batch.reshape(N, 1))
    h_pad = jnp.concatenate([h, jnp.full((NPAD - N, HID), -1e30, f32)])
    batch_pad = jnp.concatenate([batch, jnp.full((NPAD - N,), B - 1, i32)])
    mxp = _sc_maxp(h_pad, batch_pad, negB)
    return _tkF(addp, wsum, cnt, gsum, mxp)
